# Initial kernel scaffold; baseline (speedup 1.0000x reference)
import jax
import jax.numpy as jnp
from jax.experimental import pallas as pl


def kernel(atom, bond, connectivity, params):
    raise NotImplementedError("write your pallas kernel here")


# speedup vs baseline: 9.9101x; 9.9101x over previous
import jax
import jax.numpy as jnp
from jax import lax
from jax.experimental import pallas as pl
from jax.experimental.pallas import tpu as pltpu

_F = 128
_H = 8
_U = _F // _H
_NMSG = 2
_G = 8
_NEG_MIN = jnp.finfo(jnp.float32).min


def _mm(a, b):
    return lax.dot_general(a, b, (((1,), (0,)), ((), ())),
                           preferred_element_type=jnp.float32)


def _onehot_rows(idx, ncols):
    g, n = idx.shape
    col = lax.broadcasted_iota(jnp.int32, (g, n, ncols), 2)
    oh = (col == idx[:, :, None]).astype(jnp.float32)
    return oh.reshape(g * n, ncols)


def _body(aid_ref, bid_ref, src_ref, dst_ref, aemb_ref, bemb_ref, *rest):
    glob = rest[0:12]
    msg = rest[12:28]
    whead_ref, bhead_ref, out_ref = rest[28], rest[29], rest[30]

    aid = aid_ref[...]
    bid = bid_ref[...]
    g_, na = aid.shape
    nb = bid.shape[1]
    amask = (aid != 0).astype(jnp.float32)[:, :, None]
    bmask = (bid != 0).astype(jnp.float32)[:, :, None]

    atom = _mm(_onehot_rows(aid, aemb_ref.shape[0]), aemb_ref[...])
    bond = _mm(_onehot_rows(bid, bemb_ref.shape[0]), bemb_ref[...])

    E = (lax.broadcasted_iota(jnp.int32, (_H, _F), 0) ==
         lax.broadcasted_iota(jnp.int32, (_H, _F), 1) // _U).astype(jnp.float32)

    def gupd(atom, bond, gstate, wq, bq, wv, bv):
        qa = (_mm(atom, wq) + bq).reshape(g_, na, _H)
        qb = (_mm(bond, wq) + bq).reshape(g_, nb, _H)
        m = jnp.maximum(jnp.max(qa, axis=1), jnp.max(qb, axis=1))
        ea = jnp.exp(qa - m[:, None, :]) * amask
        eb = jnp.exp(qb - m[:, None, :]) * bmask
        s = jnp.sum(ea, axis=1) + jnp.sum(eb, axis=1)
        aa = (ea / s[:, None, :]).reshape(g_ * na, _H)
        ab = (eb / s[:, None, :]).reshape(g_ * nb, _H)
        va = _mm(atom, wv) + bv
        vb = _mm(bond, wv) + bv
        ctx = (jnp.sum((_mm(aa, E) * va).reshape(g_, na, _F), axis=1) +
               jnp.sum((_mm(ab, E) * vb).reshape(g_, nb, _F), axis=1))
        return ctx if gstate is None else gstate + ctx

    def gw(j):
        return (glob[4 * j][...], glob[4 * j + 1][...],
                glob[4 * j + 2][...], glob[4 * j + 3][...])

    gstate = gupd(atom, bond, None, *gw(0))

    offs = lax.broadcasted_iota(jnp.int32, (g_, nb), 0) * na
    srcg = src_ref[...] + offs
    dstg = dst_ref[...] + offs
    col = lax.broadcasted_iota(jnp.int32, (g_, nb, g_ * na), 2)
    psrc = (col == srcg[:, :, None]).astype(jnp.float32).reshape(g_ * nb, g_ * na)
    pdst = (col == dstg[:, :, None]).astype(jnp.float32).reshape(g_ * nb, g_ * na)

    for i in range(_NMSG):
        ew1, eb1, ew2, eb2, nw1, nb1, nw2, nb2 = (r[...] for r in
                                                  msg[8 * i:8 * i + 8])
        sa = _mm(psrc, atom)
        ta = _mm(pdst, atom)

        h = (_mm(bond, ew1[0:_F]) + _mm(sa, ew1[_F:2 * _F]) +
             _mm(ta, ew1[2 * _F:3 * _F]) + eb1)
        hg = _mm(gstate, ew1[3 * _F:4 * _F])
        h = jnp.maximum(h.reshape(g_, nb, 2 * _F) + hg[:, None, :], 0.0)
        bond = bond + _mm(h.reshape(g_ * nb, 2 * _F), ew2) + eb2

        h2 = _mm(sa, nw1[0:_F]) + _mm(bond, nw1[_F:2 * _F]) + nb1
        hg2 = _mm(gstate, nw1[2 * _F:3 * _F])
        h2 = jnp.maximum(h2.reshape(g_, nb, 2 * _F) + hg2[:, None, :], 0.0)
        msgs = _mm(h2.reshape(g_ * nb, 2 * _F), nw2) + nb2
        msgs = (msgs.reshape(g_, nb, _F) * bmask).reshape(g_ * nb, _F)
        agg = lax.dot_general(pdst, msgs, (((0,), (0,)), ((), ())),
                              preferred_element_type=jnp.float32)
        atom = atom + agg
        gstate = gupd(atom, bond, gstate, *gw(i + 1))

    out_ref[...] = _mm(gstate, whead_ref[...]) + bhead_ref[...]


def kernel(atom, bond, connectivity, params):
    b, a, na = atom.shape
    nb = bond.shape[2]
    bf = b * a
    aid = atom.reshape(bf, na)
    bid = bond.reshape(bf, nb)
    conn = connectivity.reshape(bf, nb, 2)
    src = conn[:, :, 1]
    dst = conn[:, :, 0]

    p = params
    args = [aid, bid, src, dst, p['atom_emb'], p['bond_emb']]
    for j in range(_NMSG + 1):
        gp = p['glob'][j]
        args += [gp['Wq'], gp['bq'].reshape(1, -1),
                 gp['Wv'], gp['bv'].reshape(1, -1)]
    for i in range(_NMSG):
        ep, npr = p['edge'][i], p['node'][i]
        args += [ep['W1'], ep['b1'].reshape(1, -1),
                 ep['W2'], ep['b2'].reshape(1, -1),
                 npr['W1'], npr['b1'].reshape(1, -1),
                 npr['W2'], npr['b2'].reshape(1, -1)]
    whead = jnp.concatenate([p['Wval'], p['Wpi']], axis=1)
    bhead = jnp.concatenate([p['bval'], p['bpi']]).reshape(1, 2)
    args += [whead, bhead]

    grid = (bf // _G,)
    id_spec = pl.BlockSpec((_G, na), lambda i: (i, 0))
    w_specs = [pl.BlockSpec(w.shape, lambda i: tuple(0 for _ in w.shape))
               for w in args[4:]]
    logits = pl.pallas_call(
        _body,
        grid=grid,
        in_specs=[id_spec, id_spec, id_spec, id_spec] + w_specs,
        out_specs=pl.BlockSpec((_G, 2), lambda i: (i, 0)),
        out_shape=jax.ShapeDtypeStruct((bf, 2), jnp.float32),
        compiler_params=pltpu.CompilerParams(
            dimension_semantics=("arbitrary",)),
    )(*args)

    value_preds = logits[:, 0].reshape(b, a)[:, 0]
    action_mask = jnp.any(atom != 0, axis=-1)
    prior = logits[:, 1].reshape(b, a)
    masked_prior = jnp.where(action_mask, prior, _NEG_MIN)[:, 1:]
    return value_preds, masked_prior


# speedup vs baseline: 11.3045x; 1.1407x over previous
import jax
import jax.numpy as jnp
from jax import lax
from jax.experimental import pallas as pl
from jax.experimental.pallas import tpu as pltpu

_F = 128
_H = 8
_U = _F // _H
_NMSG = 2
_G = 8
_NEG_MIN = jnp.finfo(jnp.float32).min


def _mm(a, b):
    return lax.dot_general(a, b, (((1,), (0,)), ((), ())),
                           preferred_element_type=jnp.float32)


def _onehot_rows(idx):
    g, n = idx.shape
    col = lax.broadcasted_iota(jnp.int32, (g, n, _F), 2)
    oh = (col == idx[:, :, None]).astype(jnp.float32)
    return oh.reshape(g * n, _F)


def _body(aid_ref, bid_ref, src_ref, dst_ref, aemb_ref, bemb_ref, *rest):
    glob = rest[0:12]
    msg = rest[12:28]
    whead_ref, bhead_ref, out_ref = rest[28], rest[29], rest[30]

    aid = aid_ref[...]
    bid = bid_ref[...]
    g_, na = aid.shape
    nb = bid.shape[1]
    amask = (aid != 0).astype(jnp.float32)[:, :, None]
    bmask = (bid != 0).astype(jnp.float32)[:, :, None]

    atom = _mm(_onehot_rows(aid), aemb_ref[...])
    bond = _mm(_onehot_rows(bid), bemb_ref[...])

    E = (lax.broadcasted_iota(jnp.int32, (_H, _F), 0) ==
         lax.broadcasted_iota(jnp.int32, (_H, _F), 1) // _U).astype(jnp.float32)

    def gupd(atom, bond, gstate, wq, bq, wv, bv):
        qa = (_mm(atom, wq) + bq).reshape(g_, na, _H)
        qb = (_mm(bond, wq) + bq).reshape(g_, nb, _H)
        m = jnp.maximum(jnp.max(qa, axis=1), jnp.max(qb, axis=1))
        ea = jnp.exp(qa - m[:, None, :]) * amask
        eb = jnp.exp(qb - m[:, None, :]) * bmask
        s = jnp.sum(ea, axis=1) + jnp.sum(eb, axis=1)
        aa = (ea / s[:, None, :]).reshape(g_ * na, _H)
        ab = (eb / s[:, None, :]).reshape(g_ * nb, _H)
        va = _mm(atom, wv) + bv
        vb = _mm(bond, wv) + bv
        ctx = (jnp.sum((_mm(aa, E) * va).reshape(g_, na, _F), axis=1) +
               jnp.sum((_mm(ab, E) * vb).reshape(g_, nb, _F), axis=1))
        return ctx if gstate is None else gstate + ctx

    def gw(j):
        return (glob[4 * j][...], glob[4 * j + 1][...],
                glob[4 * j + 2][...], glob[4 * j + 3][...])

    gstate = gupd(atom, bond, None, *gw(0))

    col = lax.broadcasted_iota(jnp.int32, (g_, nb, na), 2)
    psrc = (col == src_ref[...][:, :, None]).astype(jnp.float32)
    pdst = (col == dst_ref[...][:, :, None]).astype(jnp.float32)

    def _gather(p, x):
        r = lax.dot_general(p, x.reshape(g_, na, _F),
                            (((2,), (1,)), ((0,), (0,))),
                            preferred_element_type=jnp.float32)
        return r.reshape(g_ * nb, _F)

    for i in range(_NMSG):
        ew1, eb1, ew2, eb2, nw1, nb1, nw2, nb2 = (r[...] for r in
                                                  msg[8 * i:8 * i + 8])
        sa = _gather(psrc, atom)
        ta = _gather(pdst, atom)

        h = (_mm(bond, ew1[0:_F]) + _mm(sa, ew1[_F:2 * _F]) +
             _mm(ta, ew1[2 * _F:3 * _F]) + eb1)
        hg = _mm(gstate, ew1[3 * _F:4 * _F])
        h = jnp.maximum(h.reshape(g_, nb, 2 * _F) + hg[:, None, :], 0.0)
        bond = bond + _mm(h.reshape(g_ * nb, 2 * _F), ew2) + eb2

        h2 = _mm(sa, nw1[0:_F]) + _mm(bond, nw1[_F:2 * _F]) + nb1
        hg2 = _mm(gstate, nw1[2 * _F:3 * _F])
        h2 = jnp.maximum(h2.reshape(g_, nb, 2 * _F) + hg2[:, None, :], 0.0)
        msgs = _mm(h2.reshape(g_ * nb, 2 * _F), nw2) + nb2
        msgs = msgs.reshape(g_, nb, _F) * bmask
        agg = lax.dot_general(pdst, msgs, (((1,), (1,)), ((0,), (0,))),
                              preferred_element_type=jnp.float32)
        atom = atom + agg.reshape(g_ * na, _F)
        gstate = gupd(atom, bond, gstate, *gw(i + 1))

    out_ref[...] = _mm(gstate, whead_ref[...]) + bhead_ref[...]


def kernel(atom, bond, connectivity, params):
    b, a, na = atom.shape
    nb = bond.shape[2]
    bf = b * a
    aid = atom.reshape(bf, na)
    bid = bond.reshape(bf, nb)
    conn = connectivity.reshape(bf, nb, 2)
    src = conn[:, :, 1]
    dst = conn[:, :, 0]

    p = params
    aemb = jnp.zeros((_F, _F), jnp.float32).at[:p['atom_emb'].shape[0]].set(
        p['atom_emb'])
    bemb = jnp.zeros((_F, _F), jnp.float32).at[:p['bond_emb'].shape[0]].set(
        p['bond_emb'])
    args = [aid, bid, src, dst, aemb, bemb]
    for j in range(_NMSG + 1):
        gp = p['glob'][j]
        args += [gp['Wq'], gp['bq'].reshape(1, -1),
                 gp['Wv'], gp['bv'].reshape(1, -1)]
    for i in range(_NMSG):
        ep, npr = p['edge'][i], p['node'][i]
        args += [ep['W1'], ep['b1'].reshape(1, -1),
                 ep['W2'], ep['b2'].reshape(1, -1),
                 npr['W1'], npr['b1'].reshape(1, -1),
                 npr['W2'], npr['b2'].reshape(1, -1)]
    whead = jnp.concatenate([p['Wval'], p['Wpi']], axis=1)
    bhead = jnp.concatenate([p['bval'], p['bpi']]).reshape(1, 2)
    args += [whead, bhead]

    grid = (bf // _G,)
    id_spec = pl.BlockSpec((_G, na), lambda i: (i, 0))
    w_specs = [pl.BlockSpec(w.shape, lambda i: tuple(0 for _ in w.shape))
               for w in args[4:]]
    logits = pl.pallas_call(
        _body,
        grid=grid,
        in_specs=[id_spec, id_spec, id_spec, id_spec] + w_specs,
        out_specs=pl.BlockSpec((_G, 2), lambda i: (i, 0)),
        out_shape=jax.ShapeDtypeStruct((bf, 2), jnp.float32),
        compiler_params=pltpu.CompilerParams(
            dimension_semantics=("arbitrary",)),
    )(*args)

    value_preds = logits[:, 0].reshape(b, a)[:, 0]
    action_mask = jnp.any(atom != 0, axis=-1)
    prior = logits[:, 1].reshape(b, a)
    masked_prior = jnp.where(action_mask, prior, _NEG_MIN)[:, 1:]
    return value_preds, masked_prior


# speedup vs baseline: 15.6715x; 1.3863x over previous
import jax
import jax.numpy as jnp
from jax import lax
from jax.experimental import pallas as pl
from jax.experimental.pallas import tpu as pltpu

_F = 128
_H = 8
_U = _F // _H
_NMSG = 2
_G = 32
_NEG_MIN = jnp.finfo(jnp.float32).min


def _mm(a, b):
    return lax.dot_general(a, b, (((1,), (0,)), ((), ())),
                           preferred_element_type=jnp.float32)


def _onehot_rows(idx):
    g, n = idx.shape
    col = lax.broadcasted_iota(jnp.int32, (g, n, _F), 2)
    oh = (col == idx[:, :, None]).astype(jnp.float32)
    return oh.reshape(g * n, _F)


def _body(aid_ref, bid_ref, src_ref, dst_ref, aemb_ref, bemb_ref, *rest):
    glob = rest[0:12]
    msg = rest[12:28]
    whead_ref, bhead_ref, out_ref = rest[28], rest[29], rest[30]

    aid = aid_ref[...]
    bid = bid_ref[...]
    g_, na = aid.shape
    nb = bid.shape[1]
    amask = (aid != 0).astype(jnp.float32)[:, :, None]
    bmask = (bid != 0).astype(jnp.float32)[:, :, None]

    atom = _mm(_onehot_rows(aid), aemb_ref[...])
    bond = _mm(_onehot_rows(bid), bemb_ref[...])

    E = (lax.broadcasted_iota(jnp.int32, (_H, _F), 0) ==
         lax.broadcasted_iota(jnp.int32, (_H, _F), 1) // _U).astype(jnp.float32)

    def gupd(atom, bond, gstate, wq, bq, wv, bv):
        qa = (_mm(atom, wq) + bq).reshape(g_, na, _H)
        qb = (_mm(bond, wq) + bq).reshape(g_, nb, _H)
        m = jnp.maximum(jnp.max(qa, axis=1), jnp.max(qb, axis=1))
        ea = jnp.exp(qa - m[:, None, :]) * amask
        eb = jnp.exp(qb - m[:, None, :]) * bmask
        s = jnp.sum(ea, axis=1) + jnp.sum(eb, axis=1)
        aa = (ea / s[:, None, :]).reshape(g_ * na, _H)
        ab = (eb / s[:, None, :]).reshape(g_ * nb, _H)
        va = _mm(atom, wv) + bv
        vb = _mm(bond, wv) + bv
        ctx = (jnp.sum((_mm(aa, E) * va).reshape(g_, na, _F), axis=1) +
               jnp.sum((_mm(ab, E) * vb).reshape(g_, nb, _F), axis=1))
        return ctx if gstate is None else gstate + ctx

    def gw(j):
        return (glob[4 * j][...], glob[4 * j + 1][...],
                glob[4 * j + 2][...], glob[4 * j + 3][...])

    gstate = gupd(atom, bond, None, *gw(0))

    col = lax.broadcasted_iota(jnp.int32, (g_, nb, na), 2)
    psrc = (col == src_ref[...][:, :, None]).astype(jnp.float32)
    pdst = (col == dst_ref[...][:, :, None]).astype(jnp.float32)

    def _gather(p, x):
        r = lax.dot_general(p, x.reshape(g_, na, _F),
                            (((2,), (1,)), ((0,), (0,))),
                            preferred_element_type=jnp.float32)
        return r.reshape(g_ * nb, _F)

    for i in range(_NMSG):
        ew1, eb1, ew2, eb2, nw1, nb1, nw2, nb2 = (r[...] for r in
                                                  msg[8 * i:8 * i + 8])
        sa = _gather(psrc, atom)
        ta = _gather(pdst, atom)

        h = (_mm(bond, ew1[0:_F]) + _mm(sa, ew1[_F:2 * _F]) +
             _mm(ta, ew1[2 * _F:3 * _F]) + eb1)
        hg = _mm(gstate, ew1[3 * _F:4 * _F])
        h = jnp.maximum(h.reshape(g_, nb, 2 * _F) + hg[:, None, :], 0.0)
        bond = bond + _mm(h.reshape(g_ * nb, 2 * _F), ew2) + eb2

        h2 = _mm(sa, nw1[0:_F]) + _mm(bond, nw1[_F:2 * _F]) + nb1
        hg2 = _mm(gstate, nw1[2 * _F:3 * _F])
        h2 = jnp.maximum(h2.reshape(g_, nb, 2 * _F) + hg2[:, None, :], 0.0)
        msgs = _mm(h2.reshape(g_ * nb, 2 * _F), nw2) + nb2
        msgs = msgs.reshape(g_, nb, _F) * bmask
        agg = lax.dot_general(pdst, msgs, (((1,), (1,)), ((0,), (0,))),
                              preferred_element_type=jnp.float32)
        atom = atom + agg.reshape(g_ * na, _F)
        gstate = gupd(atom, bond, gstate, *gw(i + 1))

    out_ref[...] = _mm(gstate, whead_ref[...]) + bhead_ref[...]


def kernel(atom, bond, connectivity, params):
    b, a, na = atom.shape
    nb = bond.shape[2]
    bf = b * a
    aid = atom.reshape(bf, na)
    bid = bond.reshape(bf, nb)
    conn = connectivity.reshape(bf, nb, 2)
    src = conn[:, :, 1]
    dst = conn[:, :, 0]

    p = params
    aemb = jnp.zeros((_F, _F), jnp.float32).at[:p['atom_emb'].shape[0]].set(
        p['atom_emb'])
    bemb = jnp.zeros((_F, _F), jnp.float32).at[:p['bond_emb'].shape[0]].set(
        p['bond_emb'])
    args = [aid, bid, src, dst, aemb, bemb]
    for j in range(_NMSG + 1):
        gp = p['glob'][j]
        args += [gp['Wq'], gp['bq'].reshape(1, -1),
                 gp['Wv'], gp['bv'].reshape(1, -1)]
    for i in range(_NMSG):
        ep, npr = p['edge'][i], p['node'][i]
        args += [ep['W1'], ep['b1'].reshape(1, -1),
                 ep['W2'], ep['b2'].reshape(1, -1),
                 npr['W1'], npr['b1'].reshape(1, -1),
                 npr['W2'], npr['b2'].reshape(1, -1)]
    whead = jnp.concatenate([p['Wval'], p['Wpi']], axis=1)
    bhead = jnp.concatenate([p['bval'], p['bpi']]).reshape(1, 2)
    args += [whead, bhead]

    grid = (bf // _G,)
    id_spec = pl.BlockSpec((_G, na), lambda i: (i, 0))
    w_specs = [pl.BlockSpec(w.shape, lambda i: tuple(0 for _ in w.shape))
               for w in args[4:]]
    logits = pl.pallas_call(
        _body,
        grid=grid,
        in_specs=[id_spec, id_spec, id_spec, id_spec] + w_specs,
        out_specs=pl.BlockSpec((_G, 2), lambda i: (i, 0)),
        out_shape=jax.ShapeDtypeStruct((bf, 2), jnp.float32),
        compiler_params=pltpu.CompilerParams(
            dimension_semantics=("arbitrary",)),
    )(*args)

    value_preds = logits[:, 0].reshape(b, a)[:, 0]
    action_mask = jnp.any(atom != 0, axis=-1)
    prior = logits[:, 1].reshape(b, a)
    masked_prior = jnp.where(action_mask, prior, _NEG_MIN)[:, 1:]
    return value_preds, masked_prior
